# baseline (speedup 1.0000x reference)
import functools

import jax
import jax.numpy as jnp
from jax.experimental import pallas as pl
from jax.experimental.pallas import tpu as pltpu

_N = 10000
_E = 320000
_CIN = 128
_CH = 128
_H1, _H2, _H3 = 4, 4, 6
_COUT = 40
_G = 64


def _mm_body(x_ref, w_ref, o_ref):
    o_ref[...] = jnp.dot(x_ref[...], w_ref[...],
                         preferred_element_type=jnp.float32)


def _matmul(x, w, bn=1000):
    n, k = x.shape
    _, m = w.shape
    return pl.pallas_call(
        _mm_body,
        grid=(n // bn,),
        in_specs=[
            pl.BlockSpec((bn, k), lambda i: (i, 0)),
            pl.BlockSpec((k, m), lambda i: (0, 0)),
        ],
        out_specs=pl.BlockSpec((bn, m), lambda i: (i, 0)),
        out_shape=jax.ShapeDtypeStruct((n, m), jnp.float32),
    )(x, w)


def _gat(x, src, dst, ea_e, mean_ea, W, a_s, a_d, We, ae, b, heads, concat):
    n = x.shape[0]
    h = _matmul(x, W).reshape(n, heads, _CH)
    ssrc = (h * a_s[None]).sum(-1)
    sdst = (h * a_d[None]).sum(-1)
    we = (We.reshape(heads, _CH) * ae).sum(-1)

    logit = ssrc[src] + sdst[dst] + ea_e * we[None, :]
    logit = jnp.where(logit > 0, logit, 0.2 * logit)
    w_e = jnp.exp(logit)
    den = jax.ops.segment_sum(w_e, dst, num_segments=n)
    acc = jax.ops.segment_sum(h[src] * w_e[:, :, None], dst, num_segments=n)

    logit_s = ssrc + sdst + mean_ea * we[None, :]
    logit_s = jnp.where(logit_s > 0, logit_s, 0.2 * logit_s)
    w_s = jnp.exp(logit_s)
    den = den + w_s
    acc = acc + h * w_s[:, :, None]

    out = acc / den[:, :, None]
    if concat:
        out = out.reshape(n, heads * _CH)
    else:
        out = out.mean(axis=1)
    return out + b


def _bn(x, g, b, eps=1e-5):
    mu = x.mean(axis=0)
    var = x.var(axis=0)
    return (x - mu) / jnp.sqrt(var + eps) * g + b


def kernel(x, edge_index, edge_attr, batch, W1, as1, ad1, We1, ae1, b1, g1, bt1, W2, as2, ad2, We2, ae2, b2, g2, bt2, W3, as3, ad3, We3, ae3, b3, g3, bt3, Wh, bh):
    src = edge_index[0]
    dst = edge_index[1]
    ea_e = edge_attr
    mean_ea = jnp.mean(edge_attr)

    x1 = _bn(_gat(x, src, dst, ea_e, mean_ea, W1, as1, ad1, We1, ae1, b1, _H1, True), g1, bt1)
    x2 = _bn(_gat(x1, src, dst, ea_e, mean_ea, W2, as2, ad2, We2, ae2, b2, _H2, True), g2, bt2) + x1
    x3 = _bn(_gat(x2, src, dst, ea_e, mean_ea, W3, as3, ad3, We3, ae3, b3, _H3, False), g3, bt3)

    cnt = jax.ops.segment_sum(jnp.ones((x3.shape[0],), x3.dtype), batch, num_segments=_G)
    pooled = jax.ops.segment_sum(x3, batch, num_segments=_G) / jnp.maximum(cnt, 1.0)[:, None]
    return pooled @ Wh + bh


# speedup vs baseline: 8.8790x; 8.8790x over previous
import functools

import jax
import jax.numpy as jnp
from jax import lax
from jax.experimental import pallas as pl
from jax.experimental.pallas import tpu as pltpu
from jax.experimental.pallas import tpu_sc as plsc

_N = 10000
_E = 320000
_CIN = 128
_CH = 128
_H1, _H2, _H3 = 4, 4, 6
_COUT = 40
_G = 64

_NC = 2
_NS = 16
_EB = 80
_NSTRIP = 10
_SROWS = 1000
_ZROWS = 40


def _mm_body(x_ref, w_ref, o_ref):
    o_ref[...] = jnp.dot(x_ref[...], w_ref[...],
                         preferred_element_type=jnp.float32)


def _matmul(x, w, bn=1000):
    n, k = x.shape
    _, m = w.shape
    return pl.pallas_call(
        _mm_body,
        grid=(n // bn,),
        in_specs=[
            pl.BlockSpec((bn, k), lambda i: (i, 0)),
            pl.BlockSpec((k, m), lambda i: (0, 0)),
        ],
        out_specs=pl.BlockSpec((bn, m), lambda i: (i, 0)),
        out_shape=jax.ShapeDtypeStruct((n, m), jnp.float32),
    )(x, w)


def _make_sc_aggregate(heads):
    hpc = heads // _NC
    epb = _E // _NS
    nblk = epb // _EB

    mesh = plsc.VectorSubcoreMesh(core_axis_name="c", subcore_axis_name="s")

    @functools.partial(
        pl.kernel,
        out_type=jax.ShapeDtypeStruct((heads * _N, _CH), jnp.float32),
        mesh=mesh,
        scratch_types=[
            pltpu.VMEM((_EB,), jnp.int32),
            pltpu.VMEM((_EB,), jnp.int32),
            pltpu.VMEM((_EB,), jnp.int32),
            pltpu.VMEM((_EB,), jnp.float32),
            pltpu.VMEM((_EB, _CH), jnp.float32),
            pltpu.VMEM((_ZROWS, _CH), jnp.float32),
            pltpu.VMEM((_ZROWS, _CH), jnp.float32),
            pltpu.VMEM_SHARED((_N, _CH), jnp.float32),
            pltpu.SemaphoreType.DMA,
        ],
    )
    def agg(h_ref, src_ref, dst_ref, w_ref, out_ref,
            src_v, dst_v, gidx_v, w_v, rows_v, stage_v, flush_v, acc_sp, sem):
        c = lax.axis_index("c")
        s = lax.axis_index("s")
        ebase = s * epb

        def _zrow(i, _):
            for j in range(_CH // 16):
                stage_v[i, pl.ds(j * 16, 16)] = jnp.zeros((16,), jnp.float32)
            return _
        lax.fori_loop(0, _ZROWS, _zrow, 0)

        for hi in range(hpc):
            head = c * hpc + hi

            @pl.when(s < _NSTRIP)
            def _zero():
                for t in range(_SROWS // _ZROWS):
                    r0 = pl.multiple_of(s * _SROWS + t * _ZROWS, 8)
                    pltpu.sync_copy(stage_v, acc_sp.at[pl.ds(r0, _ZROWS)])
            plsc.subcore_barrier()

            def _blk(b, _):
                base = ebase + b * _EB
                pltpu.sync_copy(src_ref.at[pl.ds(base, _EB)], src_v)
                pltpu.sync_copy(dst_ref.at[pl.ds(base, _EB)], dst_v)
                pltpu.sync_copy(w_ref.at[pl.ds(head * _E + base, _EB)], w_v)
                for j in range(_EB // 16):
                    gidx_v[pl.ds(j * 16, 16)] = (
                        src_v[pl.ds(j * 16, 16)] + head * _N)
                pltpu.async_copy(h_ref.at[gidx_v], rows_v, sem).wait()

                def _scale(g, _c):
                    w16 = w_v[pl.ds(g * 16, 16)]
                    for t in range(16):
                        wk = w16[t]
                        row = g * 16 + t
                        for j in range(_CH // 16):
                            rows_v[row, pl.ds(j * 16, 16)] = (
                                rows_v[row, pl.ds(j * 16, 16)] * wk)
                    return _c
                lax.fori_loop(0, _EB // 16, _scale, 0)

                pltpu.sync_copy(rows_v, acc_sp.at[dst_v], add=True)
                return _
            lax.fori_loop(0, nblk, _blk, 0)
            plsc.subcore_barrier()

            @pl.when(s < _NSTRIP)
            def _flush():
                for t in range(_SROWS // _ZROWS):
                    r0 = pl.multiple_of(s * _SROWS + t * _ZROWS, 8)
                    ro = pl.multiple_of(head * _N + s * _SROWS + t * _ZROWS, 8)
                    pltpu.sync_copy(acc_sp.at[pl.ds(r0, _ZROWS)], flush_v)
                    pltpu.sync_copy(flush_v, out_ref.at[pl.ds(ro, _ZROWS)])
            plsc.subcore_barrier()

    return agg


def _gat(x, src, dst, ea_e, mean_ea, W, a_s, a_d, We, ae, b, heads, concat):
    n = x.shape[0]
    h = _matmul(x, W)
    h3 = h.reshape(n, heads, _CH)
    ssrc = (h3 * a_s[None]).sum(-1)
    sdst = (h3 * a_d[None]).sum(-1)
    we = (We.reshape(heads, _CH) * ae).sum(-1)

    logit = ssrc[src] + sdst[dst] + ea_e * we[None, :]
    logit = jnp.where(logit > 0, logit, 0.2 * logit)
    w_e = jnp.exp(logit)
    den = jax.ops.segment_sum(w_e, dst, num_segments=n)

    h_t = h3.transpose(1, 0, 2).reshape(heads * n, _CH)
    w_t = w_e.T.reshape(heads * _E)
    acc_t = _make_sc_aggregate(heads)(h_t, src, dst, w_t)
    acc = acc_t.reshape(heads, n, _CH).transpose(1, 0, 2)

    logit_s = ssrc + sdst + mean_ea * we[None, :]
    logit_s = jnp.where(logit_s > 0, logit_s, 0.2 * logit_s)
    w_s = jnp.exp(logit_s)
    den = den + w_s
    acc = acc + h3 * w_s[:, :, None]

    out = acc / den[:, :, None]
    if concat:
        out = out.reshape(n, heads * _CH)
    else:
        out = out.mean(axis=1)
    return out + b


def _bn(x, g, b, eps=1e-5):
    mu = x.mean(axis=0)
    var = x.var(axis=0)
    return (x - mu) / jnp.sqrt(var + eps) * g + b


def kernel(x, edge_index, edge_attr, batch, W1, as1, ad1, We1, ae1, b1, g1, bt1, W2, as2, ad2, We2, ae2, b2, g2, bt2, W3, as3, ad3, We3, ae3, b3, g3, bt3, Wh, bh):
    src = edge_index[0]
    dst = edge_index[1]
    ea_e = edge_attr
    mean_ea = jnp.mean(edge_attr)

    x1 = _bn(_gat(x, src, dst, ea_e, mean_ea, W1, as1, ad1, We1, ae1, b1, _H1, True), g1, bt1)
    x2 = _bn(_gat(x1, src, dst, ea_e, mean_ea, W2, as2, ad2, We2, ae2, b2, _H2, True), g2, bt2) + x1
    x3 = _bn(_gat(x2, src, dst, ea_e, mean_ea, W3, as3, ad3, We3, ae3, b3, _H3, False), g3, bt3)

    cnt = jax.ops.segment_sum(jnp.ones((x3.shape[0],), x3.dtype), batch, num_segments=_G)
    pooled = jax.ops.segment_sum(x3, batch, num_segments=_G) / jnp.maximum(cnt, 1.0)[:, None]
    return pooled @ Wh + bh


# speedup vs baseline: 11.6914x; 1.3167x over previous
import functools

import jax
import jax.numpy as jnp
from jax import lax
from jax.experimental import pallas as pl
from jax.experimental.pallas import tpu as pltpu
from jax.experimental.pallas import tpu_sc as plsc

_N = 10000
_E = 320000
_CIN = 128
_CH = 128
_H1, _H2, _H3 = 4, 4, 6
_COUT = 40
_G = 64

_NC = 2
_NS = 16
_EB = 80
_CHK = 2000
_NBLK = _CHK // _EB
_NCHK = _E // _NS // _CHK
_ZROWS = 40
_NSTRIP = 10
_SROWS = 1000


def _mm_t_body(x_ref, w_ref, o_ref):
    o_ref[...] = jnp.dot(x_ref[...], w_ref[...],
                         preferred_element_type=jnp.float32)


def _matmul_t(x, w, heads, bn=1000):
    n, k = x.shape
    nb = n // bn
    return pl.pallas_call(
        _mm_t_body,
        grid=(nb, heads),
        in_specs=[
            pl.BlockSpec((bn, k), lambda i, h: (i, 0)),
            pl.BlockSpec((k, _CH), lambda i, h: (0, h)),
        ],
        out_specs=pl.BlockSpec((bn, _CH), lambda i, h, _nb=nb: (h * _nb + i, 0)),
        out_shape=jax.ShapeDtypeStruct((heads * n, _CH), jnp.float32),
    )(x, w)


def _make_sc_aggregate(heads):
    hpc = heads // _NC
    epb = _E // _NS

    mesh = plsc.VectorSubcoreMesh(core_axis_name="c", subcore_axis_name="s")

    @functools.partial(
        pl.kernel,
        out_type=jax.ShapeDtypeStruct((heads * _N, _CH), jnp.float32),
        mesh=mesh,
        scratch_types=[
            pltpu.VMEM((_CHK,), jnp.int32),
            pltpu.VMEM((_CHK,), jnp.int32),
            pltpu.VMEM((_CHK,), jnp.float32),
            pltpu.VMEM((_EB,), jnp.int32),
            pltpu.VMEM((_EB, _CH), jnp.float32),
            pltpu.VMEM((_EB, _CH), jnp.float32),
            pltpu.VMEM((_ZROWS, _CH), jnp.float32),
            pltpu.VMEM((_ZROWS, _CH), jnp.float32),
            pltpu.VMEM_SHARED((_N, _CH), jnp.float32),
            pltpu.SemaphoreType.DMA,
            pltpu.SemaphoreType.DMA,
        ],
    )
    def agg(h_ref, src_ref, w_ref, dst_ref, out_ref,
            src_v, gidx_v, w_v, dst_v, rows_a, rows_b, stage_v, flush_v,
            acc_sp, sem_a, sem_b):
        c = lax.axis_index("c")
        s = lax.axis_index("s")

        def _zrow(i, _):
            for j in range(_CH // 16):
                stage_v[i, pl.ds(j * 16, 16)] = jnp.zeros((16,), jnp.float32)
            return _
        lax.fori_loop(0, _ZROWS, _zrow, 0)

        def _gather(blk, rows, sem):
            return pltpu.async_copy(
                h_ref.at[gidx_v.at[pl.ds(blk * _EB, _EB)]], rows, sem)

        def _proc(blk, base, rows):
            pltpu.sync_copy(dst_ref.at[pl.ds(base + blk * _EB, _EB)], dst_v)
            def _scale(g, _c2):
                w16 = w_v[pl.ds(blk * _EB + g * 16, 16)]
                for t in range(16):
                    wk = w16[t]
                    for j in range(_CH // 16):
                        rows[g * 16 + t, pl.ds(j * 16, 16)] = (
                            rows[g * 16 + t, pl.ds(j * 16, 16)] * wk)
                return _c2
            lax.fori_loop(0, _EB // 16, _scale, 0)
            pltpu.sync_copy(rows, acc_sp.at[dst_v], add=True)

        for hi in range(hpc):
            head = c * hpc + hi

            @pl.when(s < _NSTRIP)
            def _zero():
                for t in range(_SROWS // _ZROWS):
                    r0 = pl.multiple_of(s * _SROWS + t * _ZROWS, 8)
                    pltpu.sync_copy(stage_v, acc_sp.at[pl.ds(r0, _ZROWS)])
            plsc.subcore_barrier()

            def _chunk(ci, _):
                base = s * epb + ci * _CHK
                pltpu.sync_copy(src_ref.at[pl.ds(base, _CHK)], src_v)
                pltpu.sync_copy(w_ref.at[pl.ds(head * _E + base, _CHK)], w_v)

                def _gi(i, _c2):
                    gidx_v[pl.ds(i * 16, 16)] = (
                        src_v[pl.ds(i * 16, 16)] + head * _N)
                    return _c2
                lax.fori_loop(0, _CHK // 16, _gi, 0)

                _gather(0, rows_a, sem_a).wait()
                cp_b = _gather(1, rows_b, sem_b)

                def _pair(p, _c2):
                    blk_a = 2 * p
                    _proc(blk_a, base, rows_a)
                    @pl.when(blk_a + 2 < _NBLK)
                    def _ga():
                        _gather(blk_a + 2, rows_a, sem_a)
                    pltpu.make_async_copy(
                        h_ref.at[gidx_v.at[pl.ds(0, _EB)]], rows_b,
                        sem_b).wait()
                    _proc(blk_a + 1, base, rows_b)
                    @pl.when(blk_a + 3 < _NBLK)
                    def _gb():
                        _gather(blk_a + 3, rows_b, sem_b)
                    @pl.when(blk_a + 2 < _NBLK)
                    def _wa():
                        pltpu.make_async_copy(
                            h_ref.at[gidx_v.at[pl.ds(0, _EB)]], rows_a,
                            sem_a).wait()
                    return _c2
                lax.fori_loop(0, (_NBLK - 1) // 2, _pair, 0)
                _proc(_NBLK - 1, base, rows_a)
                return _
            lax.fori_loop(0, _NCHK, _chunk, 0)
            plsc.subcore_barrier()

            @pl.when(s < _NSTRIP)
            def _flush():
                for t in range(_SROWS // _ZROWS):
                    r0 = pl.multiple_of(s * _SROWS + t * _ZROWS, 8)
                    ro = pl.multiple_of(head * _N + s * _SROWS + t * _ZROWS, 8)
                    pltpu.sync_copy(acc_sp.at[pl.ds(r0, _ZROWS)], flush_v)
                    pltpu.sync_copy(flush_v, out_ref.at[pl.ds(ro, _ZROWS)])
            plsc.subcore_barrier()

    return agg


def _gat(x, src, dst, ea_e, mean_ea, W, a_s, a_d, We, ae, b, heads, concat):
    n = x.shape[0]
    h_t = _matmul_t(x, W, heads)
    h3 = h_t.reshape(heads, n, _CH)
    ssrc = jnp.einsum("hnc,hc->nh", h3, a_s)
    sdst = jnp.einsum("hnc,hc->nh", h3, a_d)
    we = (We.reshape(heads, _CH) * ae).sum(-1)

    logit = ssrc[src] + sdst[dst] + ea_e * we[None, :]
    logit = jnp.where(logit > 0, logit, 0.2 * logit)
    w_e = jnp.exp(logit)
    den = jax.ops.segment_sum(w_e, dst, num_segments=n)

    w_t = w_e.T.reshape(heads * _E)
    acc_t = _make_sc_aggregate(heads)(h_t, src, w_t, dst)
    acc3 = acc_t.reshape(heads, n, _CH)

    logit_s = ssrc + sdst + mean_ea * we[None, :]
    logit_s = jnp.where(logit_s > 0, logit_s, 0.2 * logit_s)
    w_s = jnp.exp(logit_s)
    den = den + w_s
    outm = (acc3 + h3 * w_s.T[:, :, None]) / den.T[:, :, None]
    if concat:
        out = outm.transpose(1, 0, 2).reshape(n, heads * _CH)
    else:
        out = outm.mean(axis=0)
    return out + b


def _bn(x, g, b, eps=1e-5):
    mu = x.mean(axis=0)
    var = x.var(axis=0)
    return (x - mu) / jnp.sqrt(var + eps) * g + b


def kernel(x, edge_index, edge_attr, batch, W1, as1, ad1, We1, ae1, b1, g1, bt1, W2, as2, ad2, We2, ae2, b2, g2, bt2, W3, as3, ad3, We3, ae3, b3, g3, bt3, Wh, bh):
    src = edge_index[0]
    dst = edge_index[1]
    ea_e = edge_attr
    mean_ea = jnp.mean(edge_attr)

    x1 = _bn(_gat(x, src, dst, ea_e, mean_ea, W1, as1, ad1, We1, ae1, b1, _H1, True), g1, bt1)
    x2 = _bn(_gat(x1, src, dst, ea_e, mean_ea, W2, as2, ad2, We2, ae2, b2, _H2, True), g2, bt2) + x1
    x3 = _bn(_gat(x2, src, dst, ea_e, mean_ea, W3, as3, ad3, We3, ae3, b3, _H3, False), g3, bt3)

    cnt = jax.ops.segment_sum(jnp.ones((x3.shape[0],), x3.dtype), batch, num_segments=_G)
    pooled = jax.ops.segment_sum(x3, batch, num_segments=_G) / jnp.maximum(cnt, 1.0)[:, None]
    return pooled @ Wh + bh
